# baseline (speedup 1.0000x reference)
import functools

import jax
import jax.numpy as jnp
import numpy as np
from jax import lax
from jax.experimental import pallas as pl
from jax.experimental.pallas import tpu as pltpu
from jax.experimental.pallas import tpu_sc as plsc

_IN_FEATURES = 4096
_IN_CHANNELS = 8
_OUT_CHANNELS = 4
_WINDOW = 64
_STRIDES = 32
_NUM_P = (_IN_FEATURES - _WINDOW) // _STRIDES
_NK = _NUM_P * _OUT_CHANNELS
_B = 128

_NW = 32
_POS_PER_W = 4
_SPAN = 256
_SLAB_B = 16
_NSLAB = _B // _SLAB_B
_LANES = 16
_ROWW = _IN_CHANNELS * _SPAN + 1
_WQ = 128


def _sc_body(x_hbm, w_hbm, out_hbm, xb0, xb1, obuf, wspm, wsm0, wsm1,
             sem0, sem1, wsem0, wsem1):
    wid = (lax.axis_index("s") * 2 + lax.axis_index("c")).astype(jnp.int32)

    xbase = pl.multiple_of(128 * jnp.minimum(wid, _NW - 2), 128)
    orow0 = pl.multiple_of(_LANES * wid, 8)

    lane = lax.iota(jnp.int32, _LANES)
    xbufs = (xb0, xb1)
    sems = (sem0, sem1)
    wsms = (wsm0, wsm1)
    wsems = (wsem0, wsem1)

    @pl.when(lax.axis_index("s") == 0)
    def _():
        pltpu.sync_copy(w_hbm, wspm)

    plsc.subcore_barrier()

    def lo_of(i):
        p = _POS_PER_W * wid + i
        valid = p < _NUM_P
        gstart = _STRIDES * p + _STRIDES * (p == _NUM_P - 1).astype(jnp.int32)
        return jnp.where(valid, gstart - xbase, 0)

    def wrow_of(i):
        return jnp.clip(_OUT_CHANNELS * (_POS_PER_W * wid + i), 0,
                        _NK - _OUT_CHANNELS)

    def wfill(i, q, buf):
        return pltpu.make_async_copy(
            wspm.at[pl.ds(wrow_of(i), _OUT_CHANNELS), pl.ds(q * _WQ, _WQ)],
            wsms[buf], wsems[buf])

    def slab_copy(s, buf):
        return [pltpu.make_async_copy(
            x_hbm.at[pl.ds(s * _SLAB_B, _SLAB_B), k, pl.ds(xbase, _SPAN)],
            xbufs[buf].at[:, pl.ds(k * _SPAN, _SPAN)], sems[buf])
            for k in range(_IN_CHANNELS)]

    for c in slab_copy(0, 0):
        c.start()

    def slab_pair(s0, carry):
        for b in range(2):
            s = s0 * 2 + b
            for c in slab_copy(s, b):
                c.wait()

            @pl.when(s + 1 < _NSLAB)
            def _():
                for c in slab_copy(s + 1, 1 - b):
                    c.start()

            xb = xbufs[b]
            zeros = jnp.zeros((_LANES,), jnp.float32)
            wfill(0, 0, 0).start()

            def i_body(i, icarry):
                lo_i = lo_of(i)
                accs = (zeros,) * (2 * _OUT_CHANNELS)
                for q in range(4):
                    buf = q % 2
                    wfill(i, q, buf).wait()

                    @pl.when(_POS_PER_W * i + q + 1 < 16)
                    def _():
                        wfill(i + (1 if q == 3 else 0), (q + 1) % 4,
                              (q + 1) % 2).start()

                    wsm = wsms[buf]

                    @plsc.parallel_loop(0, _WQ // _LANES, unroll=2,
                                        carry=accs)
                    def accs(c8, accs):
                        accs = list(accs)
                        c16 = q * (_WQ // _LANES) + c8
                        kq = c16 // (_WINDOW // _LANES)
                        xcol0 = kq * (_SPAN - _WINDOW) + c16 * _LANES + lo_i
                        cv0 = jnp.full((_LANES,), xcol0, jnp.int32)
                        col0 = c8 * _LANES
                        for u in range(_LANES):
                            xv = plsc.load_gather(xb, [lane, cv0 + u])
                            for o in range(_OUT_CHANNELS):
                                h = 2 * o + (u & 1)
                                accs[h] = accs[h] + xv * wsm[o, col0 + u]
                        return tuple(accs)

                col = s * _SLAB_B + lane
                for o in range(_OUT_CHANNELS):
                    plsc.store_scatter(
                        obuf,
                        [jnp.full((_LANES,), _OUT_CHANNELS * i + o, jnp.int32),
                         col],
                        accs[2 * o] + accs[2 * o + 1])
                return icarry

            lax.fori_loop(0, _POS_PER_W, i_body, 0)
        return carry

    lax.fori_loop(0, _NSLAB // 2, slab_pair, 0)

    pltpu.sync_copy(obuf, out_hbm.at[pl.ds(orow0, _LANES), :])


@jax.jit
def kernel(x, weight):
    mesh = plsc.VectorSubcoreMesh(core_axis_name="c", subcore_axis_name="s")
    run = pl.kernel(
        _sc_body,
        mesh=mesh,
        compiler_params=pltpu.CompilerParams(
            use_tc_tiling_on_sc=False, needs_layout_passes=False),
        out_type=jax.ShapeDtypeStruct((_LANES * _NW, _B), jnp.float32),
        scratch_types=[
            pltpu.VMEM((_SLAB_B, _ROWW), jnp.float32),
            pltpu.VMEM((_SLAB_B, _ROWW), jnp.float32),
            pltpu.VMEM((_LANES, _B), jnp.float32),
            pltpu.VMEM_SHARED((_NK, _IN_CHANNELS * _WINDOW), jnp.float32),
            pltpu.SMEM((_OUT_CHANNELS, _WQ), jnp.float32),
            pltpu.SMEM((_OUT_CHANNELS, _WQ), jnp.float32),
            pltpu.SemaphoreType.DMA,
            pltpu.SemaphoreType.DMA,
            pltpu.SemaphoreType.DMA,
            pltpu.SemaphoreType.DMA,
        ],
    )
    out = run(x, weight)
    return out[:_NK].T


# speedup vs baseline: 1.7460x; 1.7460x over previous
import jax
import jax.numpy as jnp
from jax import lax
from jax.experimental import pallas as pl
from jax.experimental.pallas import tpu as pltpu
from jax.experimental.pallas import tpu_sc as plsc

_IN_FEATURES = 4096
_IN_CHANNELS = 8
_OUT_CHANNELS = 4
_WINDOW = 64
_STRIDES = 32
_NUM_P = (_IN_FEATURES - _WINDOW) // _STRIDES
_NK = _NUM_P * _OUT_CHANNELS
_B = 128

_NW = 32
_POS_PER_W = 4
_SPAN = 256
_SLAB_B = 16
_NSLAB = _B // _SLAB_B
_LANES = 16
_ROWW = _IN_CHANNELS * _SPAN + 1
_WQ = 128


def _sc_body(x_hbm, w_hbm, out_hbm, xb0, xb1, obuf, wspm, wsm0, wsm1,
             sem0, sem1, wsem0, wsem1):
    wid = (lax.axis_index("s") * 2 + lax.axis_index("c")).astype(jnp.int32)

    xbase = pl.multiple_of(128 * jnp.minimum(wid, _NW - 2), 128)
    orow0 = pl.multiple_of(_LANES * wid, 8)

    lane = lax.iota(jnp.int32, _LANES)
    xbufs = (xb0, xb1)
    sems = (sem0, sem1)
    wsms = (wsm0, wsm1)
    wsems = (wsem0, wsem1)

    @pl.when(lax.axis_index("s") == 0)
    def _():
        pltpu.sync_copy(w_hbm, wspm)

    plsc.subcore_barrier()

    def lo_of(i):
        p = _POS_PER_W * wid + i
        valid = p < _NUM_P
        gstart = _STRIDES * p + _STRIDES * (p == _NUM_P - 1).astype(jnp.int32)
        return jnp.where(valid, gstart - xbase, 0)

    def wrow_of(i):
        return jnp.clip(_OUT_CHANNELS * (_POS_PER_W * wid + i), 0,
                        _NK - _OUT_CHANNELS)

    def wfill(i, q, buf):
        return pltpu.make_async_copy(
            wspm.at[pl.ds(wrow_of(i), _OUT_CHANNELS), pl.ds(q * _WQ, _WQ)],
            wsms[buf], wsems[buf])

    def slab_copy(s, buf):
        return [pltpu.make_async_copy(
            x_hbm.at[pl.ds(s * _SLAB_B, _SLAB_B), k, pl.ds(xbase, _SPAN)],
            xbufs[buf].at[:, pl.ds(k * _SPAN, _SPAN)], sems[buf])
            for k in range(_IN_CHANNELS)]

    for c in slab_copy(0, 0):
        c.start()

    def slab_pair(s0, carry):
        for b in range(2):
            s = s0 * 2 + b
            for c in slab_copy(s, b):
                c.wait()

            @pl.when(s + 1 < _NSLAB)
            def _():
                for c in slab_copy(s + 1, 1 - b):
                    c.start()

            xb = xbufs[b]
            zeros = jnp.zeros((_LANES,), jnp.float32)
            wfill(0, 0, 0).start()

            def i_body(i, icarry):
                lo_i = lo_of(i)
                accs = (zeros,) * (2 * _OUT_CHANNELS)
                for q in range(4):
                    buf = q % 2
                    wfill(i, q, buf).wait()

                    @pl.when(_POS_PER_W * i + q + 1 < 16)
                    def _():
                        wfill(i + (1 if q == 3 else 0), (q + 1) % 4,
                              (q + 1) % 2).start()

                    wsm = wsms[buf]

                    @plsc.parallel_loop(0, _WQ // _LANES, carry=accs)
                    def accs(c8, accs):
                        accs = list(accs)
                        c16 = q * (_WQ // _LANES) + c8
                        kq = c16 // (_WINDOW // _LANES)
                        xcol0 = kq * (_SPAN - _WINDOW) + c16 * _LANES + lo_i
                        cv0 = jnp.full((_LANES,), xcol0, jnp.int32)
                        col0 = c8 * _LANES
                        for u in range(_LANES):
                            xv = plsc.load_gather(xb, [lane, cv0 + u])
                            for o in range(_OUT_CHANNELS):
                                h = 2 * o + (u & 1)
                                accs[h] = accs[h] + xv * wsm[o, col0 + u]
                        return tuple(accs)

                col = s * _SLAB_B + lane
                for o in range(_OUT_CHANNELS):
                    plsc.store_scatter(
                        obuf,
                        [jnp.full((_LANES,), _OUT_CHANNELS * i + o, jnp.int32),
                         col],
                        accs[2 * o] + accs[2 * o + 1])
                return icarry

            lax.fori_loop(0, _POS_PER_W, i_body, 0)
        return carry

    lax.fori_loop(0, _NSLAB // 2, slab_pair, 0)

    pltpu.sync_copy(obuf, out_hbm.at[pl.ds(orow0, _LANES), :])


@jax.jit
def kernel(x, weight):
    mesh = plsc.VectorSubcoreMesh(core_axis_name="c", subcore_axis_name="s")
    run = pl.kernel(
        _sc_body,
        mesh=mesh,
        compiler_params=pltpu.CompilerParams(
            use_tc_tiling_on_sc=False, needs_layout_passes=False),
        out_type=jax.ShapeDtypeStruct((_LANES * _NW, _B), jnp.float32),
        scratch_types=[
            pltpu.VMEM((_SLAB_B, _ROWW), jnp.float32),
            pltpu.VMEM((_SLAB_B, _ROWW), jnp.float32),
            pltpu.VMEM((_LANES, _B), jnp.float32),
            pltpu.VMEM_SHARED((_NK, _IN_CHANNELS * _WINDOW), jnp.float32),
            pltpu.SMEM((_OUT_CHANNELS, _WQ), jnp.float32),
            pltpu.SMEM((_OUT_CHANNELS, _WQ), jnp.float32),
            pltpu.SemaphoreType.DMA,
            pltpu.SemaphoreType.DMA,
            pltpu.SemaphoreType.DMA,
            pltpu.SemaphoreType.DMA,
        ],
    )
    out = run(x, weight)
    return out[:_NK].T
